# initial kernel scaffold (unmeasured)
import jax
import jax.numpy as jnp
from jax import lax
from jax.experimental import pallas as pl
from jax.experimental.pallas import tpu as pltpu

N_DEV = 8
SQ = 256
SKV = 4096
HQ_PER = 8
DH = 128
D_MODEL = 1024
D_SLICE = HQ_PER * DH
SCALE = 0.08838834764831843
BLK = 64


def kernel(x, Wq, K_ext, V_ext, Wo):
    def body(x_ref, wq_hbm, k_ref, v_ref, wo_hbm, out_ref,
             wq_v, wo_v, q_buf, ctx_buf, send_buf, recv_buf,
             local_sems, send_sems, recv_sems):
        my_pos = lax.axis_index("i")

        wq_dma = pltpu.make_async_copy(
            wq_hbm.at[:, pl.ds(my_pos * D_SLICE, D_SLICE)], wq_v,
            local_sems.at[0])
        wo_dma = pltpu.make_async_copy(
            wo_hbm.at[pl.ds(my_pos * D_SLICE, D_SLICE), :], wo_v,
            local_sems.at[1])
        wq_dma.start()
        wo_dma.start()

        qb = lax.broadcasted_iota(jnp.int32, (SQ, SKV), 0) // BLK
        kb = lax.broadcasted_iota(jnp.int32, (SQ, SKV), 1) // BLK
        mask = (qb == kb) | (kb == 0) | (((qb + kb) % 3) == 0)

        wq_dma.wait()
        q = jnp.dot(x_ref[0].astype(jnp.bfloat16),
                    wq_v[...].astype(jnp.bfloat16),
                    preferred_element_type=jnp.float32)
        q_buf[...] = q.astype(jnp.bfloat16)

        for h in range(HQ_PER):
            qh = q_buf[:, h * DH:(h + 1) * DH]
            kh = k_ref[0, :, h, :].astype(jnp.bfloat16)
            scores = lax.dot_general(
                qh, kh, (((1,), (1,)), ((), ())),
                preferred_element_type=jnp.float32) * SCALE
            scores = jnp.where(mask, scores, -1e9)
            m = jnp.max(scores, axis=1, keepdims=True)
            w = jnp.exp(scores - m)
            s = jnp.sum(w, axis=1, keepdims=True)
            vh = v_ref[0, :, h, :].astype(jnp.bfloat16)
            ctx = lax.dot_general(
                w.astype(jnp.bfloat16), vh, (((1,), (0,)), ((), ())),
                preferred_element_type=jnp.float32)
            ctx_buf[:, h * DH:(h + 1) * DH] = (ctx / s).astype(jnp.bfloat16)

        wo_dma.wait()
        acc = jnp.dot(ctx_buf[...], wo_v[...].astype(jnp.bfloat16),
                      preferred_element_type=jnp.float32)

        for step, dist in enumerate((1, 2, 4)):
            partner = my_pos ^ dist
            send_buf[...] = acc.astype(jnp.bfloat16)
            rdma = pltpu.make_async_remote_copy(
                src_ref=send_buf,
                dst_ref=recv_buf.at[step],
                send_sem=send_sems.at[step],
                recv_sem=recv_sems.at[step],
                device_id=(partner,),
                device_id_type=pl.DeviceIdType.MESH,
            )
            rdma.start()
            rdma.wait()
            acc = acc + recv_buf[step].astype(jnp.float32)

        out_ref[0] = acc

    return pl.pallas_call(
        body,
        out_shape=jax.ShapeDtypeStruct((1, SQ, D_MODEL), jnp.float32),
        in_specs=[
            pl.BlockSpec(memory_space=pltpu.VMEM),
            pl.BlockSpec(memory_space=pltpu.ANY),
            pl.BlockSpec(memory_space=pltpu.VMEM),
            pl.BlockSpec(memory_space=pltpu.VMEM),
            pl.BlockSpec(memory_space=pltpu.ANY),
        ],
        out_specs=pl.BlockSpec(memory_space=pltpu.VMEM),
        scratch_shapes=[
            pltpu.VMEM((D_MODEL, D_SLICE), jnp.float32),
            pltpu.VMEM((D_SLICE, D_MODEL), jnp.float32),
            pltpu.VMEM((SQ, D_SLICE), jnp.bfloat16),
            pltpu.VMEM((SQ, D_SLICE), jnp.bfloat16),
            pltpu.VMEM((SQ, D_MODEL), jnp.bfloat16),
            pltpu.VMEM((3, SQ, D_MODEL), jnp.bfloat16),
            pltpu.SemaphoreType.DMA((2,)),
            pltpu.SemaphoreType.DMA((3,)),
            pltpu.SemaphoreType.DMA((3,)),
        ],
    )(x, Wq, K_ext, V_ext, Wo)


# baseline (device time: 107084 ns/iter reference)
import jax
import jax.numpy as jnp
from jax import lax
from jax.experimental import pallas as pl
from jax.experimental.pallas import tpu as pltpu

N_DEV = 8
SQ = 256
SKV = 4096
HQ_PER = 8
DH = 128
D_MODEL = 1024
D_SLICE = HQ_PER * DH
SCALE = 0.08838834764831843
BLK = 64


def kernel(x, Wq, K_ext, V_ext, Wo):
    def body(x_ref, wq_hbm, k_ref, v_ref, wo_hbm, out_ref,
             wq_v, wo_v, q_buf, ctx_buf, send_buf, recv_buf,
             local_sems, send_sems, recv_sems):
        my_pos = lax.axis_index("i")

        wq_dma = pltpu.make_async_copy(
            wq_hbm.at[:, pl.ds(my_pos * D_SLICE, D_SLICE)], wq_v,
            local_sems.at[0])
        wo_dma = pltpu.make_async_copy(
            wo_hbm.at[pl.ds(my_pos * D_SLICE, D_SLICE), :], wo_v,
            local_sems.at[1])
        wq_dma.start()
        wo_dma.start()

        qb = lax.broadcasted_iota(jnp.int32, (SQ, SKV), 0) // BLK
        kb = lax.broadcasted_iota(jnp.int32, (SQ, SKV), 1) // BLK
        mask = (qb == kb) | (kb == 0) | (((qb + kb) % 3) == 0)

        wq_dma.wait()
        q = jnp.dot(x_ref[0].astype(jnp.bfloat16),
                    wq_v[...].astype(jnp.bfloat16),
                    preferred_element_type=jnp.float32)
        q_buf[...] = q.astype(jnp.bfloat16)

        for h in range(HQ_PER):
            qh = q_buf[:, h * DH:(h + 1) * DH]
            kh = k_ref[0, :, h, :].astype(jnp.bfloat16)
            scores = lax.dot_general(
                qh, kh, (((1,), (1,)), ((), ())),
                preferred_element_type=jnp.float32) * SCALE
            scores = jnp.where(mask, scores, -1e9)
            m = jnp.max(scores, axis=1, keepdims=True)
            w = jnp.exp(scores - m)
            s = jnp.sum(w, axis=1, keepdims=True)
            vh = v_ref[0, :, h, :].astype(jnp.bfloat16)
            ctx = lax.dot_general(
                w.astype(jnp.bfloat16), vh, (((1,), (0,)), ((), ())),
                preferred_element_type=jnp.float32)
            ctx_buf[:, h * DH:(h + 1) * DH] = (ctx / s).astype(jnp.bfloat16)

        wo_dma.wait()
        acc = jnp.dot(ctx_buf[...], wo_v[...].astype(jnp.bfloat16),
                      preferred_element_type=jnp.float32)

        for step, dist in enumerate((1, 2, 4)):
            partner = my_pos ^ dist
            send_buf[...] = acc.astype(jnp.bfloat16)
            rdma = pltpu.make_async_remote_copy(
                src_ref=send_buf,
                dst_ref=recv_buf.at[step],
                send_sem=send_sems.at[step],
                recv_sem=recv_sems.at[step],
                device_id=(partner,),
                device_id_type=pl.DeviceIdType.MESH,
            )
            rdma.start()
            rdma.wait()
            acc = acc + recv_buf[step].astype(jnp.float32)

        out_ref[0] = acc

    return pl.pallas_call(
        body,
        out_shape=jax.ShapeDtypeStruct((1, SQ, D_MODEL), jnp.float32),
        in_specs=[
            pl.BlockSpec(memory_space=pltpu.VMEM),
            pl.BlockSpec(memory_space=pl.ANY),
            pl.BlockSpec(memory_space=pltpu.VMEM),
            pl.BlockSpec(memory_space=pltpu.VMEM),
            pl.BlockSpec(memory_space=pl.ANY),
        ],
        out_specs=pl.BlockSpec(memory_space=pltpu.VMEM),
        scratch_shapes=[
            pltpu.VMEM((D_MODEL, D_SLICE), jnp.float32),
            pltpu.VMEM((D_SLICE, D_MODEL), jnp.float32),
            pltpu.VMEM((SQ, D_SLICE), jnp.bfloat16),
            pltpu.VMEM((SQ, D_SLICE), jnp.bfloat16),
            pltpu.VMEM((SQ, D_MODEL), jnp.bfloat16),
            pltpu.VMEM((3, SQ, D_MODEL), jnp.bfloat16),
            pltpu.SemaphoreType.DMA((2,)),
            pltpu.SemaphoreType.DMA((3,)),
            pltpu.SemaphoreType.DMA((3,)),
        ],
        compiler_params=pltpu.CompilerParams(
            vmem_limit_bytes=100 * 1024 * 1024,
        ),
    )(x, Wq, K_ext, V_ext, Wo)


# device time: 43430 ns/iter; 2.4657x vs baseline; 2.4657x over previous
import os

import jax
import jax.numpy as jnp
from jax import lax
from jax.experimental import pallas as pl
from jax.experimental.pallas import tpu as pltpu

N_DEV = 8
SQ = 256
SKV = 4096
HQ_PER = 8
DH = 128
D_MODEL = 1024
D_SLICE = HQ_PER * DH
SCALE = 0.08838834764831843
BLK = 64
XOR_STEPS = (1, 3, 4)


def kernel(x, Wq, K_ext, V_ext, Wo):
    no_comm = os.environ.get("KERNEL_NO_COMM") == "1"
    no_compute = os.environ.get("KERNEL_NO_COMPUTE") == "1"

    def body(x_ref, wq_hbm, k_ref, v_ref, wo_hbm, out_ref,
             wq_v, wo_v, q_buf, ctx_buf, ones_buf, send_buf, recv_buf,
             local_sems, send_sems, recv_sems):
        my_pos = lax.axis_index("i")

        wq_dma = pltpu.make_async_copy(
            wq_hbm.at[:, pl.ds(my_pos * D_SLICE, D_SLICE)], wq_v,
            local_sems.at[0])
        wo_dma = pltpu.make_async_copy(
            wo_hbm.at[pl.ds(my_pos * D_SLICE, D_SLICE), :], wo_v,
            local_sems.at[1])
        wq_dma.start()
        wo_dma.start()

        if not no_comm:
            barrier_sem = pltpu.get_barrier_semaphore()
            for dist in XOR_STEPS:
                pl.semaphore_signal(
                    barrier_sem, inc=1,
                    device_id=(my_pos ^ dist,),
                    device_id_type=pl.DeviceIdType.MESH)
            pl.semaphore_wait(barrier_sem, len(XOR_STEPS))

        if no_compute:
            wq_dma.wait()
            wo_dma.wait()
            acc = x_ref[0]
        else:
            qb = lax.broadcasted_iota(jnp.int32, (SQ, SKV), 0) // BLK
            kb = lax.broadcasted_iota(jnp.int32, (SQ, SKV), 1) // BLK
            mask = (qb == kb) | (kb == 0) | (((qb + kb) % 3) == 0)
            bias = jnp.where(mask, 0.0, -1e9)
            ones_buf[...] = jnp.ones((SKV, DH), jnp.bfloat16)

            wq_dma.wait()
            q = jnp.dot(x_ref[0].astype(jnp.bfloat16),
                        wq_v[...].astype(jnp.bfloat16),
                        preferred_element_type=jnp.float32)
            q_buf[...] = (q * SCALE).astype(jnp.bfloat16)

            for h in range(HQ_PER):
                qh = q_buf[:, h * DH:(h + 1) * DH]
                kh = k_ref[0, :, h, :].astype(jnp.bfloat16)
                scores = lax.dot_general(
                    qh, kh, (((1,), (1,)), ((), ())),
                    preferred_element_type=jnp.float32)
                w = jnp.exp(scores + bias).astype(jnp.bfloat16)
                s = lax.dot_general(
                    w, ones_buf[...], (((1,), (0,)), ((), ())),
                    preferred_element_type=jnp.float32)
                vh = v_ref[0, :, h, :].astype(jnp.bfloat16)
                ctx = lax.dot_general(
                    w, vh, (((1,), (0,)), ((), ())),
                    preferred_element_type=jnp.float32)
                ctx_buf[:, h * DH:(h + 1) * DH] = (
                    ctx / s[:, 0:1]).astype(jnp.bfloat16)

            wo_dma.wait()
            acc = jnp.dot(ctx_buf[...], wo_v[...].astype(jnp.bfloat16),
                          preferred_element_type=jnp.float32)

        if not no_comm:
            for step, dist in enumerate(XOR_STEPS):
                partner = my_pos ^ dist
                send_buf[...] = acc.astype(jnp.bfloat16)
                rdma = pltpu.make_async_remote_copy(
                    src_ref=send_buf,
                    dst_ref=recv_buf.at[step],
                    send_sem=send_sems.at[step],
                    recv_sem=recv_sems.at[step],
                    device_id=(partner,),
                    device_id_type=pl.DeviceIdType.MESH,
                )
                rdma.start()
                rdma.wait()
                acc = acc + recv_buf[step].astype(jnp.float32)

        out_ref[0] = acc

    return pl.pallas_call(
        body,
        out_shape=jax.ShapeDtypeStruct((1, SQ, D_MODEL), jnp.float32),
        in_specs=[
            pl.BlockSpec(memory_space=pltpu.VMEM),
            pl.BlockSpec(memory_space=pl.ANY),
            pl.BlockSpec(memory_space=pltpu.VMEM),
            pl.BlockSpec(memory_space=pltpu.VMEM),
            pl.BlockSpec(memory_space=pl.ANY),
        ],
        out_specs=pl.BlockSpec(memory_space=pltpu.VMEM),
        scratch_shapes=[
            pltpu.VMEM((D_MODEL, D_SLICE), jnp.float32),
            pltpu.VMEM((D_SLICE, D_MODEL), jnp.float32),
            pltpu.VMEM((SQ, D_SLICE), jnp.bfloat16),
            pltpu.VMEM((SQ, D_SLICE), jnp.bfloat16),
            pltpu.VMEM((SKV, DH), jnp.bfloat16),
            pltpu.VMEM((SQ, D_MODEL), jnp.bfloat16),
            pltpu.VMEM((3, SQ, D_MODEL), jnp.bfloat16),
            pltpu.SemaphoreType.DMA((2,)),
            pltpu.SemaphoreType.DMA((3,)),
            pltpu.SemaphoreType.DMA((3,)),
        ],
        compiler_params=pltpu.CompilerParams(
            vmem_limit_bytes=100 * 1024 * 1024,
            collective_id=0,
        ),
    )(x, Wq, K_ext, V_ext, Wo)
